# baseline (speedup 1.0000x reference)
import jax
import jax.numpy as jnp
from jax.experimental import pallas as pl
from jax.experimental.pallas import tpu as pltpu

D_MODEL = 2048
K_CODES = 512
CODE_DIM = 64
TAU = 1.0

_BR = 512
_BM = 1024
_NQ = D_MODEL // _BR


def _fused_block(z_ref, ct2_ref, ca2_ref, x_ref, o_ref, w_ref):
    i = pl.program_id(0)

    @pl.when(i < _NQ)
    def _quantize():
        ct2 = ct2_ref[...]
        c22 = (0.25 / TAU) * jnp.sum(ct2 * ct2, axis=0, keepdims=True)
        ct2b = (ct2 * (1.0 / TAU)).astype(jnp.bfloat16)
        ca2 = ca2_ref[...]
        base = i * _BR
        for u in range(D_MODEL // 128):
            zp = z_ref[:, 128 * u:128 * (u + 1)].astype(jnp.bfloat16)
            l = jnp.dot(zp, ct2b, preferred_element_type=jnp.float32) - c22
            e = jnp.exp(l.astype(jnp.bfloat16))
            qs = jnp.dot(e, ca2, preferred_element_type=jnp.float32)
            q0 = qs[:, :CODE_DIM] / qs[:, CODE_DIM:CODE_DIM + 1]
            q1 = qs[:, 128:128 + CODE_DIM] / qs[:, 128 + CODE_DIM:128 + CODE_DIM + 1]
            w_ref[pl.ds(base, _BR), 128 * u:128 * (u + 1)] = (
                jnp.concatenate([q0, q1], axis=1))

    @pl.when(i >= _NQ)
    def _gemm():
        o_ref[...] = jnp.dot(x_ref[...], w_ref[...],
                             preferred_element_type=jnp.float32)


def kernel(x, subspace_params, centroids):
    z2 = subspace_params.reshape(D_MODEL, D_MODEL)

    ct = 2.0 * centroids.T
    ct2 = jnp.zeros((2 * CODE_DIM, 2 * K_CODES), jnp.float32)
    ct2 = ct2.at[:CODE_DIM, :K_CODES].set(ct).at[CODE_DIM:, K_CODES:].set(ct)
    ca = jnp.zeros((K_CODES, 2 * CODE_DIM), jnp.float32)
    ca = ca.at[:, :CODE_DIM].set(centroids).at[:, CODE_DIM].set(1.0)
    ca2 = jnp.zeros((2 * K_CODES, 4 * CODE_DIM), jnp.float32)
    ca2 = ca2.at[:K_CODES, :2 * CODE_DIM].set(ca)
    ca2 = ca2.at[K_CODES:, 2 * CODE_DIM:].set(ca).astype(jnp.bfloat16)

    m = x.shape[0]
    nsteps = _NQ + m // _BM
    out = pl.pallas_call(
        _fused_block,
        grid=(nsteps,),
        in_specs=[
            pl.BlockSpec((_BR, D_MODEL),
                         lambda i: (jnp.minimum(i, _NQ - 1), 0)),
            pl.BlockSpec((2 * CODE_DIM, 2 * K_CODES), lambda i: (0, 0)),
            pl.BlockSpec((2 * K_CODES, 4 * CODE_DIM), lambda i: (0, 0)),
            pl.BlockSpec((_BM, D_MODEL),
                         lambda i: (jnp.maximum(i - _NQ, 0), 0)),
        ],
        out_specs=pl.BlockSpec((_BM, D_MODEL),
                               lambda i: (jnp.maximum(i - _NQ, 0), 0)),
        out_shape=jax.ShapeDtypeStruct((m, D_MODEL), jnp.float32),
        scratch_shapes=[pltpu.VMEM((D_MODEL, D_MODEL), jnp.float32)],
    )(z2, ct2, ca2, x)
    return out


# speedup vs baseline: 1.0017x; 1.0017x over previous
import jax
import jax.numpy as jnp
from jax.experimental import pallas as pl
from jax.experimental.pallas import tpu as pltpu

D_MODEL = 2048
K_CODES = 512
CODE_DIM = 64
TAU = 1.0

_BR = 512
_BM = 512
_NQ = D_MODEL // _BR


def _fused_block(z_ref, ct2_ref, ca2_ref, x_ref, o_ref, w_ref):
    i = pl.program_id(0)

    @pl.when(i < _NQ)
    def _quantize():
        ct2 = ct2_ref[...]
        c22 = (0.25 / TAU) * jnp.sum(ct2 * ct2, axis=0, keepdims=True)
        ct2b = (ct2 * (1.0 / TAU)).astype(jnp.bfloat16)
        ca2 = ca2_ref[...]
        base = i * _BR
        for u in range(D_MODEL // 128):
            zp = z_ref[:, 128 * u:128 * (u + 1)].astype(jnp.bfloat16)
            l = jnp.dot(zp, ct2b, preferred_element_type=jnp.float32) - c22
            e = jnp.exp(l.astype(jnp.bfloat16))
            qs = jnp.dot(e, ca2, preferred_element_type=jnp.float32)
            q0 = qs[:, :CODE_DIM] / qs[:, CODE_DIM:CODE_DIM + 1]
            q1 = qs[:, 128:128 + CODE_DIM] / qs[:, 128 + CODE_DIM:128 + CODE_DIM + 1]
            w_ref[pl.ds(base, _BR), 128 * u:128 * (u + 1)] = (
                jnp.concatenate([q0, q1], axis=1))

    @pl.when(i >= _NQ)
    def _gemm():
        o_ref[...] = jnp.dot(x_ref[...], w_ref[...],
                             preferred_element_type=jnp.float32)


def kernel(x, subspace_params, centroids):
    z2 = subspace_params.reshape(D_MODEL, D_MODEL)

    ct = 2.0 * centroids.T
    ct2 = jnp.zeros((2 * CODE_DIM, 2 * K_CODES), jnp.float32)
    ct2 = ct2.at[:CODE_DIM, :K_CODES].set(ct).at[CODE_DIM:, K_CODES:].set(ct)
    ca = jnp.zeros((K_CODES, 2 * CODE_DIM), jnp.float32)
    ca = ca.at[:, :CODE_DIM].set(centroids).at[:, CODE_DIM].set(1.0)
    ca2 = jnp.zeros((2 * K_CODES, 4 * CODE_DIM), jnp.float32)
    ca2 = ca2.at[:K_CODES, :2 * CODE_DIM].set(ca)
    ca2 = ca2.at[K_CODES:, 2 * CODE_DIM:].set(ca).astype(jnp.bfloat16)

    m = x.shape[0]
    nsteps = _NQ + m // _BM
    out = pl.pallas_call(
        _fused_block,
        grid=(nsteps,),
        in_specs=[
            pl.BlockSpec((_BR, D_MODEL),
                         lambda i: (jnp.minimum(i, _NQ - 1), 0)),
            pl.BlockSpec((2 * CODE_DIM, 2 * K_CODES), lambda i: (0, 0)),
            pl.BlockSpec((2 * K_CODES, 4 * CODE_DIM), lambda i: (0, 0)),
            pl.BlockSpec((_BM, D_MODEL),
                         lambda i: (jnp.maximum(i - _NQ, 0), 0)),
        ],
        out_specs=pl.BlockSpec((_BM, D_MODEL),
                               lambda i: (jnp.maximum(i - _NQ, 0), 0)),
        out_shape=jax.ShapeDtypeStruct((m, D_MODEL), jnp.float32),
        scratch_shapes=[pltpu.VMEM((D_MODEL, D_MODEL), jnp.float32)],
    )(z2, ct2, ca2, x)
    return out
